# baseline (speedup 1.0000x reference)
import functools
import math

import jax
import jax.numpy as jnp
from jax import lax
from jax.experimental import pallas as pl
from jax.experimental.pallas import tpu as pltpu
from jax.experimental.pallas import tpu_sc as plsc

D_MODEL = 512
SCALE = math.sqrt(float(D_MODEL))


@functools.cache
def _make_sc_embed(V, D, R, W):
    info = plsc.get_sparse_core_info()
    NC, NS, L = info.num_cores, info.num_subcores, info.num_lanes
    NW = NC * NS
    B = R * W
    assert B % NW == 0
    b_per_w = B // NW
    w_per_row = W // b_per_w
    CH = 64
    NB = 3
    assert b_per_w % CH == 0
    NCHUNK = b_per_w // CH

    mesh = plsc.VectorSubcoreMesh(core_axis_name="c", subcore_axis_name="s")

    @functools.partial(
        pl.kernel,
        mesh=mesh,
        out_type=jax.ShapeDtypeStruct((R, W, D), jnp.float32),
        scratch_types=[
            pltpu.VMEM((b_per_w,), jnp.int32),
            pltpu.VMEM((CH, D), jnp.float32),
            pltpu.VMEM((CH, D), jnp.float32),
            pltpu.VMEM((CH, D), jnp.float32),
            pltpu.SemaphoreType.DMA,
            pltpu.SemaphoreType.DMA,
            pltpu.SemaphoreType.DMA,
            pltpu.SemaphoreType.DMA,
            pltpu.SemaphoreType.DMA,
            pltpu.SemaphoreType.DMA,
        ],
    )
    def k(idx_hbm, table_hbm, out_hbm, idx_v,
          buf0, buf1, buf2, gs0, gs1, gs2, os0, os1, os2):
        wid = lax.axis_index("s") * NC + lax.axis_index("c")
        row = wid // w_per_row
        col = (wid % w_per_row) * b_per_w
        pltpu.sync_copy(idx_hbm.at[row, pl.ds(col, b_per_w)], idx_v)

        bufs = (buf0, buf1, buf2)
        gsems = (gs0, gs1, gs2)
        osems = (os0, os1, os2)

        def start_gather(c):
            return pltpu.async_copy(
                table_hbm.at[idx_v.at[pl.ds(c * CH, CH)]],
                bufs[c % NB], gsems[c % NB])

        def start_out(c):
            return pltpu.async_copy(
                bufs[c % NB], out_hbm.at[row, pl.ds(col + c * CH, CH)],
                osems[c % NB])

        gh = [None] * NCHUNK
        oh = [None] * NCHUNK
        for c in range(NB):
            gh[c] = start_gather(c)
        for c in range(NCHUNK):
            if c >= 1 and c + NB - 1 < NCHUNK:
                oh[c - 1].wait()
                gh[c + NB - 1] = start_gather(c + NB - 1)
            gh[c].wait()
            buf = bufs[c % NB]

            def scale_row(r, carry, buf=buf):
                for j in range(D // L):
                    buf[r, pl.ds(j * L, L)] = buf[r, pl.ds(j * L, L)] * SCALE
                return carry

            lax.fori_loop(0, CH, scale_row, 0)
            oh[c] = start_out(c)
        for c in range(NCHUNK - NB, NCHUNK):
            oh[c].wait()

    return k


def kernel(x, lut):
    R, W = x.shape
    V, D = lut.shape
    k = _make_sc_embed(V, D, R, W)
    return k(x, lut)


# speedup vs baseline: 1.0584x; 1.0584x over previous
import functools
import math

import jax
import jax.numpy as jnp
from jax import lax
from jax.experimental import pallas as pl
from jax.experimental.pallas import tpu as pltpu
from jax.experimental.pallas import tpu_sc as plsc

D_MODEL = 512
SCALE = math.sqrt(float(D_MODEL))


@functools.cache
def _make_sc_embed(V, D, R, W):
    info = plsc.get_sparse_core_info()
    NC, NS, L = info.num_cores, info.num_subcores, info.num_lanes
    NW = NC * NS
    B = R * W
    assert B % NW == 0
    b_per_w = B // NW
    w_per_row = W // b_per_w
    CH = 32
    NB = 4
    assert b_per_w % CH == 0
    NCHUNK = b_per_w // CH
    assert NCHUNK % NB == 0
    NG = NCHUNK // NB

    mesh = plsc.VectorSubcoreMesh(core_axis_name="c", subcore_axis_name="s")

    @functools.partial(
        pl.kernel,
        mesh=mesh,
        out_type=jax.ShapeDtypeStruct((R, W, D), jnp.float32),
        scratch_types=[
            pltpu.VMEM((b_per_w,), jnp.int32),
            pltpu.VMEM((CH, D), jnp.float32),
            pltpu.VMEM((CH, D), jnp.float32),
            pltpu.VMEM((CH, D), jnp.float32),
            pltpu.VMEM((CH, D), jnp.float32),
            pltpu.SemaphoreType.DMA,
            pltpu.SemaphoreType.DMA,
            pltpu.SemaphoreType.DMA,
            pltpu.SemaphoreType.DMA,
            pltpu.SemaphoreType.DMA,
            pltpu.SemaphoreType.DMA,
            pltpu.SemaphoreType.DMA,
            pltpu.SemaphoreType.DMA,
        ],
    )
    def k(idx_hbm, table_hbm, out_hbm, idx_v,
          buf0, buf1, buf2, buf3, gs0, gs1, gs2, gs3, os0, os1, os2, os3):
        wid = lax.axis_index("s") * NC + lax.axis_index("c")
        row = wid // w_per_row
        col = (wid % w_per_row) * b_per_w
        pltpu.sync_copy(idx_hbm.at[row, pl.ds(col, b_per_w)], idx_v)

        bufs = (buf0, buf1, buf2, buf3)
        gsems = (gs0, gs1, gs2, gs3)
        osems = (os0, os1, os2, os3)

        def gather_desc(c, b):
            off = pl.multiple_of(c * CH, CH)
            return pltpu.make_async_copy(
                table_hbm.at[idx_v.at[pl.ds(off, CH)]], bufs[b], gsems[b])

        def out_desc(c, b):
            off = pl.multiple_of(col + c * CH, CH)
            return pltpu.make_async_copy(
                bufs[b], out_hbm.at[row, pl.ds(off, CH)], osems[b])

        for b in range(NB):
            gather_desc(b, b).start()

        def chunk_body(g, c, b):
            def drain_and_prefetch():
                out_desc(c - 1, (b + 3) % NB).wait()
                gather_desc(c + 3, (b + 3) % NB).start()

            if b == 0:
                @pl.when(g >= 1)
                def _():
                    drain_and_prefetch()
            else:
                out_desc(c - 1, (b + 3) % NB).wait()

                @pl.when(g < NG - 1)
                def _():
                    gather_desc(c + 3, (b + 3) % NB).start()

            gather_desc(c, b).wait()
            buf = bufs[b]

            def scale_row(r, carry):
                for j in range(D // L):
                    buf[r, pl.ds(j * L, L)] = buf[r, pl.ds(j * L, L)] * SCALE
                return carry

            lax.fori_loop(0, CH, scale_row, 0)
            out_desc(c, b).start()

        def group_body(g, carry):
            for b in range(NB):
                chunk_body(g, g * NB + b, b)
            return carry

        lax.fori_loop(0, NG, group_body, 0)
        out_desc(NCHUNK - 1, (NCHUNK - 1) % NB).wait()

    return k


def kernel(x, lut):
    R, W = x.shape
    V, D = lut.shape
    k = _make_sc_embed(V, D, R, W)
    return k(x, lut)
